# baseline (speedup 1.0000x reference)
import functools

import jax
import jax.numpy as jnp
from jax import lax
from jax.experimental import pallas as pl
from jax.experimental.pallas import tpu as pltpu
from jax.experimental.pallas import tpu_sc as plsc

VOCAB = 100000
D = 128
B = 1024
L = 200
N = B * L
NC, NS = 2, 16
NW = NC * NS
PER_W = N // NW
CHUNK = 256
NCHUNK = PER_W // CHUNK
NBUF = 3
MAIN = ((NCHUNK - NBUF) // NBUF) * NBUF
NOUT = MAIN // NBUF

_mesh = plsc.VectorSubcoreMesh(core_axis_name="c", subcore_axis_name="s")


@functools.partial(
    pl.kernel,
    mesh=_mesh,
    out_type=jax.ShapeDtypeStruct((N, D), jnp.float32),
    scratch_types=[pltpu.VMEM((PER_W,), jnp.int32),
                   pltpu.VMEM((NBUF, CHUNK, D), jnp.float32)]
                  + [pltpu.SemaphoreType.DMA] * (2 * NBUF),
)
def _gather(w_hbm, idx_hbm, out_hbm, idx_v, rows_v, *sems):
    gsems, osems = sems[:NBUF], sems[NBUF:]
    wid = lax.axis_index("s") * NC + lax.axis_index("c")
    base = wid * PER_W
    pltpu.sync_copy(idx_hbm.at[pl.ds(base, PER_W)], idx_v)

    def gath(g, b):
        return pltpu.make_async_copy(
            w_hbm.at[idx_v.at[pl.ds(pl.multiple_of(g * CHUNK, CHUNK), CHUNK)]],
            rows_v.at[b], gsems[b])

    def outc(g, b):
        return pltpu.make_async_copy(
            rows_v.at[b], out_hbm.at[pl.ds(base + g * CHUNK, CHUNK)],
            osems[b])

    for b in range(NBUF):
        gath(b, b).start()

    def outer(t, carry):
        go = t * NBUF
        for b in range(NBUF):
            gath(go + b, b).wait()
            outc(go + b, b).start()
        for b in range(NBUF):
            outc(go + b, b).wait()
            gath(go + NBUF + b, b).start()
        return carry

    lax.fori_loop(0, NOUT, outer, 0)

    drained = set()
    for g in range(MAIN, NCHUNK):
        b = g % NBUF
        if g >= MAIN + NBUF:
            outc(g - NBUF, b).wait()
            drained.add(g - NBUF)
            gath(g, b).start()
        gath(g, b).wait()
        outc(g, b).start()
    for g in range(MAIN, NCHUNK):
        if g not in drained:
            outc(g, g % NBUF).wait()


def kernel(x, target, text_inputs, W):
    idx = x.reshape(-1).astype(jnp.int32)
    out = _gather(W, idx)
    return out.reshape(B, L, D)


# speedup vs baseline: 1.0450x; 1.0450x over previous
import functools

import jax
import jax.numpy as jnp
from jax import lax
from jax.experimental import pallas as pl
from jax.experimental.pallas import tpu as pltpu
from jax.experimental.pallas import tpu_sc as plsc

VOCAB = 100000
D = 128
B = 1024
L = 200
N = B * L
NC, NS = 2, 16
NW = NC * NS
PER_W = N // NW
CHUNK = 128
NCHUNK = PER_W // CHUNK
NBUF = 7
MAIN = ((NCHUNK - NBUF) // NBUF) * NBUF
NOUT = MAIN // NBUF

_mesh = plsc.VectorSubcoreMesh(core_axis_name="c", subcore_axis_name="s")


@functools.partial(
    pl.kernel,
    mesh=_mesh,
    out_type=jax.ShapeDtypeStruct((N, D), jnp.float32),
    scratch_types=[pltpu.VMEM((PER_W,), jnp.int32),
                   pltpu.VMEM((NBUF, CHUNK, D), jnp.float32)]
                  + [pltpu.SemaphoreType.DMA] * (2 * NBUF),
)
def _gather(w_hbm, idx_hbm, out_hbm, idx_v, rows_v, *sems):
    gsems, osems = sems[:NBUF], sems[NBUF:]
    wid = lax.axis_index("s") * NC + lax.axis_index("c")
    base = wid * PER_W
    pltpu.sync_copy(idx_hbm.at[pl.ds(base, PER_W)], idx_v)

    def gath(g, b):
        return pltpu.make_async_copy(
            w_hbm.at[idx_v.at[pl.ds(pl.multiple_of(g * CHUNK, CHUNK), CHUNK)]],
            rows_v.at[b], gsems[b])

    def outc(g, b):
        return pltpu.make_async_copy(
            rows_v.at[b], out_hbm.at[pl.ds(base + g * CHUNK, CHUNK)],
            osems[b])

    for b in range(NBUF):
        gath(b, b).start()

    def outer(t, carry):
        go = t * NBUF
        for b in range(NBUF):
            gath(go + b, b).wait()
            outc(go + b, b).start()
        for b in range(NBUF):
            outc(go + b, b).wait()
            gath(go + NBUF + b, b).start()
        return carry

    lax.fori_loop(0, NOUT, outer, 0)

    drained = set()
    for g in range(MAIN, NCHUNK):
        b = g % NBUF
        if g >= MAIN + NBUF:
            outc(g - NBUF, b).wait()
            drained.add(g - NBUF)
            gath(g, b).start()
        gath(g, b).wait()
        outc(g, b).start()
    for g in range(MAIN, NCHUNK):
        if g not in drained:
            outc(g, g % NBUF).wait()


def kernel(x, target, text_inputs, W):
    idx = x.reshape(-1).astype(jnp.int32)
    out = _gather(W, idx)
    return out.reshape(B, L, D)


# speedup vs baseline: 1.0710x; 1.0249x over previous
import functools

import jax
import jax.numpy as jnp
from jax import lax
from jax.experimental import pallas as pl
from jax.experimental.pallas import tpu as pltpu
from jax.experimental.pallas import tpu_sc as plsc

VOCAB = 100000
D = 128
B = 1024
L = 200
N = B * L
NC, NS = 2, 16
NW = NC * NS
PER_W = N // NW
CHUNK = 128
NCHUNK = PER_W // CHUNK
NBUF = 7
LOOKAHEAD = 4
LAG = NBUF - LOOKAHEAD

_mesh = plsc.VectorSubcoreMesh(core_axis_name="c", subcore_axis_name="s")


@functools.partial(
    pl.kernel,
    mesh=_mesh,
    out_type=jax.ShapeDtypeStruct((N, D), jnp.float32),
    scratch_types=[pltpu.VMEM((PER_W,), jnp.int32),
                   pltpu.VMEM((NBUF, CHUNK, D), jnp.float32)]
                  + [pltpu.SemaphoreType.DMA] * (2 * NBUF),
)
def _gather(w_hbm, idx_hbm, out_hbm, idx_v, rows_v, *sems):
    gsems, osems = sems[:NBUF], sems[NBUF:]
    wid = lax.axis_index("s") * NC + lax.axis_index("c")
    base = wid * PER_W
    pltpu.sync_copy(idx_hbm.at[pl.ds(base, PER_W)], idx_v)

    def gath(g, b):
        return pltpu.make_async_copy(
            w_hbm.at[idx_v.at[pl.ds(pl.multiple_of(g * CHUNK, CHUNK), CHUNK)]],
            rows_v.at[b], gsems[b])

    def outc(g, b):
        return pltpu.make_async_copy(
            rows_v.at[b], out_hbm.at[pl.ds(base + g * CHUNK, CHUNK)],
            osems[b])

    def pstep(g, r):
        gath(g, r).wait()
        outc(g, r).start()

    def prefire(g, r2):
        outc(g - LAG, r2).wait()
        gath(g + LOOKAHEAD, r2).start()

    for g in range(LOOKAHEAD):
        gath(g, g % NBUF).start()
    for g in range(LOOKAHEAD):
        pstep(g, g % NBUF)
        if g - LAG >= 0:
            prefire(g, (g + LOOKAHEAD) % NBUF)
        else:
            gath(g + LOOKAHEAD, (g + LOOKAHEAD) % NBUF).start()

    FIRST = LOOKAHEAD
    ROUNDS = (NCHUNK - 2 * LOOKAHEAD) // NBUF
    MID_END = FIRST + ROUNDS * NBUF

    def outer(t, carry):
        go = FIRST + t * NBUF
        for j in range(NBUF):
            pstep(go + j, (FIRST + j) % NBUF)
            prefire(go + j, (FIRST + j + LOOKAHEAD) % NBUF)
        return carry

    lax.fori_loop(0, ROUNDS, outer, 0)

    for g in range(MID_END, NCHUNK):
        pstep(g, g % NBUF)
        outc(g - LAG, (g - LAG) % NBUF).wait()
    for g in range(NCHUNK - LAG, NCHUNK):
        outc(g, g % NBUF).wait()


def kernel(x, target, text_inputs, W):
    idx = x.reshape(-1).astype(jnp.int32)
    out = _gather(W, idx)
    return out.reshape(B, L, D)


# speedup vs baseline: 1.0759x; 1.0046x over previous
import functools

import jax
import jax.numpy as jnp
from jax import lax
from jax.experimental import pallas as pl
from jax.experimental.pallas import tpu as pltpu
from jax.experimental.pallas import tpu_sc as plsc

VOCAB = 100000
D = 128
B = 1024
L = 200
N = B * L
NC, NS = 2, 16
NW = NC * NS
PER_W = N // NW
CHUNK = 64
NCHUNK = PER_W // CHUNK
NBUF = 14
LOOKAHEAD = 8
LAG = NBUF - LOOKAHEAD

_mesh = plsc.VectorSubcoreMesh(core_axis_name="c", subcore_axis_name="s")


@functools.partial(
    pl.kernel,
    mesh=_mesh,
    out_type=jax.ShapeDtypeStruct((N, D), jnp.float32),
    scratch_types=[pltpu.VMEM((PER_W,), jnp.int32),
                   pltpu.VMEM((NBUF, CHUNK, D), jnp.float32)]
                  + [pltpu.SemaphoreType.DMA] * (2 * NBUF),
)
def _gather(w_hbm, idx_hbm, out_hbm, idx_v, rows_v, *sems):
    gsems, osems = sems[:NBUF], sems[NBUF:]
    wid = lax.axis_index("s") * NC + lax.axis_index("c")
    base = wid * PER_W
    pltpu.sync_copy(idx_hbm.at[pl.ds(base, PER_W)], idx_v)

    def gath(g, b):
        return pltpu.make_async_copy(
            w_hbm.at[idx_v.at[pl.ds(pl.multiple_of(g * CHUNK, CHUNK), CHUNK)]],
            rows_v.at[b], gsems[b])

    def outc(g, b):
        return pltpu.make_async_copy(
            rows_v.at[b], out_hbm.at[pl.ds(base + g * CHUNK, CHUNK)],
            osems[b])

    def pstep(g, r):
        gath(g, r).wait()
        outc(g, r).start()

    def prefire(g, r2):
        outc(g - LAG, r2).wait()
        gath(g + LOOKAHEAD, r2).start()

    for g in range(LOOKAHEAD):
        gath(g, g % NBUF).start()
    for g in range(LOOKAHEAD):
        pstep(g, g % NBUF)
        if g - LAG >= 0:
            prefire(g, (g + LOOKAHEAD) % NBUF)
        else:
            gath(g + LOOKAHEAD, (g + LOOKAHEAD) % NBUF).start()

    FIRST = LOOKAHEAD
    ROUNDS = (NCHUNK - 2 * LOOKAHEAD) // NBUF
    MID_END = FIRST + ROUNDS * NBUF

    def outer(t, carry):
        go = FIRST + t * NBUF
        for j in range(NBUF):
            pstep(go + j, (FIRST + j) % NBUF)
            prefire(go + j, (FIRST + j + LOOKAHEAD) % NBUF)
        return carry

    lax.fori_loop(0, ROUNDS, outer, 0)

    for g in range(MID_END, NCHUNK):
        pstep(g, g % NBUF)
        outc(g - LAG, (g - LAG) % NBUF).wait()
    for g in range(NCHUNK - LAG, NCHUNK):
        outc(g, g % NBUF).wait()


def kernel(x, target, text_inputs, W):
    idx = x.reshape(-1).astype(jnp.int32)
    out = _gather(W, idx)
    return out.reshape(B, L, D)
